# Initial kernel scaffold; baseline (speedup 1.0000x reference)
import jax
import jax.numpy as jnp
from jax.experimental import pallas as pl


def kernel(input1, input2, embedding_table):
    raise NotImplementedError("write your pallas kernel here")


# speedup vs baseline: 12.6787x; 12.6787x over previous
import functools

import jax
import jax.numpy as jnp
from jax import lax
from jax.experimental import pallas as pl
from jax.experimental.pallas import tpu as pltpu
from jax.experimental.pallas import tpu_sc as plsc

_B = 4096
_L = 20
_D = 4
_LANES = 16
_NC = 2
_NS = 16
_NW = _NC * _NS
_BPW = _B // _NW
_G = _BPW // _LANES
_TBL_ROWS = 32


def _sc_body(a_hbm, b_hbm, tbl_hbm, out_hbm, a_v, b_v, tbl_v, row_v, out_v):
    wid = lax.axis_index("s") * _NC + lax.axis_index("c")
    pltpu.sync_copy(a_hbm.at[wid], a_v)
    pltpu.sync_copy(b_hbm.at[wid], b_v)
    pltpu.sync_copy(tbl_hbm.at[pl.ds(0, _TBL_ROWS * _D)], tbl_v)
    lane = lax.broadcasted_iota(jnp.int32, (_LANES,), 0)

    for g in range(_G):
        bt = [b_v[pl.ds((g * _L + j) * _LANES, _LANES)] for j in range(_L)]
        for j in range(_L + 1):
            row_v[pl.ds(j * _LANES, _LANES)] = jnp.full((_LANES,), j, jnp.int32)

        def i_body(i, carry, g=g, bt=bt):
            a_off = pl.multiple_of((g * _L + i) * _LANES, _LANES)
            ai = a_v[pl.ds(a_off, _LANES)]
            diag = row_v[pl.ds(0, _LANES)]
            cur = diag + 1
            row_v[pl.ds(0, _LANES)] = cur
            for j in range(1, _L + 1):
                up = row_v[pl.ds(j * _LANES, _LANES)]
                sub = jnp.where(ai == bt[j - 1], diag, diag + 1)
                cur = jnp.minimum(jnp.minimum(up, cur) + 1, sub)
                row_v[pl.ds(j * _LANES, _LANES)] = cur
                diag = up
            return carry

        lax.fori_loop(0, _L, i_body, 0)

        dist = jnp.minimum(row_v[pl.ds(_L * _LANES, _LANES)], _TBL_ROWS - 1)
        base = dist * _D
        obase = (g * _LANES * _D) + lane * _D
        for c in range(_D):
            col = plsc.load_gather(tbl_v, [base + c])
            plsc.store_scatter(out_v, [obase + c], col)

    pltpu.sync_copy(out_v, out_hbm.at[pl.ds(wid * _BPW * _D, _BPW * _D)])


_sc_call = functools.partial(
    pl.kernel,
    mesh=plsc.VectorSubcoreMesh(core_axis_name="c", subcore_axis_name="s"),
    out_type=jax.ShapeDtypeStruct((_B * _D,), jnp.float32),
    compiler_params=pltpu.CompilerParams(needs_layout_passes=False),
    scratch_types=[
        pltpu.VMEM((_G * _L * _LANES,), jnp.int32),
        pltpu.VMEM((_G * _L * _LANES,), jnp.int32),
        pltpu.VMEM((_TBL_ROWS * _D,), jnp.float32),
        pltpu.VMEM(((_L + 1) * _LANES,), jnp.int32),
        pltpu.VMEM((_BPW * _D,), jnp.float32),
    ],
)(_sc_body)


def kernel(input1, input2, embedding_table):
    a = input1.reshape(_NW, _G, _LANES, _L).transpose(0, 1, 3, 2).reshape(_NW, -1)
    b = input2.reshape(_NW, _G, _LANES, _L).transpose(0, 1, 3, 2).reshape(_NW, -1)
    out = _sc_call(a, b, embedding_table.reshape(-1))
    return out.reshape(_B, _D)


# speedup vs baseline: 16.7805x; 1.3235x over previous
import functools

import jax
import jax.numpy as jnp
from jax import lax
from jax.experimental import pallas as pl
from jax.experimental.pallas import tpu as pltpu
from jax.experimental.pallas import tpu_sc as plsc

_B = 4096
_L = 20
_D = 4
_LANES = 16
_NC = 2
_NS = 16
_NW = _NC * _NS
_BPW = _B // _NW
_G = _BPW // _LANES
_GI = 4
_NTOK = 128
_TBL_ROWS = 32

_MASKM = (1 << _L) - 1


def _splat(v):
    return jnp.full((_LANES,), v, jnp.int32)


def _sc_body(a_hbm, b_hbm, tbl_hbm, out_hbm, a_v, b_v, tbl_v, peq_v, out_v):
    wid = lax.axis_index("s") * _NC + lax.axis_index("c")
    pltpu.sync_copy(a_hbm.at[wid], a_v)
    pltpu.sync_copy(b_hbm.at[wid], b_v)
    pltpu.sync_copy(tbl_hbm.at[pl.ds(0, _TBL_ROWS * _D)], tbl_v)
    lane = lax.broadcasted_iota(jnp.int32, (_LANES,), 0)

    def zero_body(i, carry):
        off = pl.multiple_of(i * _LANES, _LANES)
        for t in range(_GI):
            peq_v[pl.ds(off + t * _NTOK * _LANES, _LANES)] = _splat(0)
        return carry

    lax.fori_loop(0, _NTOK, zero_body, 0)

    for half in range(_G // _GI):
        groups = [half * _GI + t for t in range(_GI)]

        for t, g in enumerate(groups):
            tb = t * _NTOK * _LANES
            for j in range(_L):
                bj = b_v[pl.ds((g * _L + j) * _LANES, _LANES)]
                idx = tb + (bj << 4) + lane
                plsc.addupdate_scatter(peq_v, [idx], _splat(1 << j))

        def i_body(i, carry, groups=groups):
            vps, vns, scs = carry
            nvps, nvns, nscs = [], [], []
            for t, g in enumerate(groups):
                vp, vn, sc = vps[t], vns[t], scs[t]
                a_off = pl.multiple_of((g * _L + i) * _LANES, _LANES)
                ai = a_v[pl.ds(a_off, _LANES)]
                peq = plsc.load_gather(
                    peq_v, [t * _NTOK * _LANES + (ai << 4) + lane])
                x = peq | vn
                d0 = (((vp + (x & vp)) & _MASKM) ^ vp) | x
                hn = vp & d0
                hp = vn | ((vp | d0) ^ _MASKM)
                sc = sc + (hp >> (_L - 1)) - (hn >> (_L - 1))
                xs = ((hp << 1) | 1) & _MASKM
                nvns.append(xs & d0)
                nvps.append(((hn << 1) & _MASKM) | ((xs | d0) ^ _MASKM))
                nscs.append(sc)
            return (tuple(nvps), tuple(nvns), tuple(nscs))

        init = (tuple(_splat(_MASKM) for _ in groups),
                tuple(_splat(0) for _ in groups),
                tuple(_splat(_L) for _ in groups))
        _, _, scores = lax.fori_loop(0, _L, i_body, init)

        if half + 1 < _G // _GI:
            for t, g in enumerate(groups):
                tb = t * _NTOK * _LANES
                for j in range(_L):
                    bj = b_v[pl.ds((g * _L + j) * _LANES, _LANES)]
                    plsc.store_scatter(peq_v, [tb + (bj << 4) + lane], _splat(0))

        for t, g in enumerate(groups):
            dist = jnp.minimum(scores[t], _TBL_ROWS - 1)
            base = dist * _D
            obase = (g * _LANES * _D) + lane * _D
            for c in range(_D):
                col = plsc.load_gather(tbl_v, [base + c])
                plsc.store_scatter(out_v, [obase + c], col)

    pltpu.sync_copy(out_v, out_hbm.at[pl.ds(wid * _BPW * _D, _BPW * _D)])


_sc_call = functools.partial(
    pl.kernel,
    mesh=plsc.VectorSubcoreMesh(core_axis_name="c", subcore_axis_name="s"),
    out_type=jax.ShapeDtypeStruct((_B * _D,), jnp.float32),
    compiler_params=pltpu.CompilerParams(needs_layout_passes=False),
    scratch_types=[
        pltpu.VMEM((_G * _L * _LANES,), jnp.int32),
        pltpu.VMEM((_G * _L * _LANES,), jnp.int32),
        pltpu.VMEM((_TBL_ROWS * _D,), jnp.float32),
        pltpu.VMEM((_GI * _NTOK * _LANES,), jnp.int32),
        pltpu.VMEM((_BPW * _D,), jnp.float32),
    ],
)(_sc_body)


def kernel(input1, input2, embedding_table):
    a = input1.reshape(_NW, _G, _LANES, _L).transpose(0, 1, 3, 2).reshape(_NW, -1)
    b = input2.reshape(_NW, _G, _LANES, _L).transpose(0, 1, 3, 2).reshape(_NW, -1)
    out = _sc_call(a, b, embedding_table.reshape(-1))
    return out.reshape(_B, _D)
